# baseline (speedup 1.0000x reference)
import jax
import jax.numpy as jnp
from jax.experimental import pallas as pl
from jax.experimental.pallas import tpu as pltpu

_N = 200000
_STRIDE = 3000
_NIDS = 64
_CHUNKS1 = 8
_C1 = _N // _CHUNKS1


def _dma_kernel(x_ref, o_ref, zbuf, sems, zsems):
    zbuf[...] = jnp.zeros_like(zbuf)
    copies = []
    n = 0
    for c in range(_CHUNKS1):
        cp = pltpu.make_async_copy(
            x_ref.at[1, pl.ds(c * _C1, _C1), :],
            o_ref.at[1, pl.ds(c * _C1, _C1), :],
            sems.at[n])
        cp.start()
        copies.append(cp)
        n += 1
    for k in range(_NIDS - 1):
        s = _STRIDE * k + 1
        cp = pltpu.make_async_copy(
            x_ref.at[0, pl.ds(s, _STRIDE - 1), :],
            o_ref.at[0, pl.ds(s, _STRIDE - 1), :],
            sems.at[n])
        cp.start()
        copies.append(cp)
        n += 1
    tail_start = _STRIDE * (_NIDS - 1) + 1
    cp = pltpu.make_async_copy(
        x_ref.at[0, pl.ds(tail_start, _N - tail_start), :],
        o_ref.at[0, pl.ds(tail_start, _N - tail_start), :],
        sems.at[n])
    cp.start()
    copies.append(cp)
    n += 1
    zcopies = []
    for k in range(_NIDS):
        cp = pltpu.make_async_copy(
            zbuf.at[pl.ds(0, 1), :],
            o_ref.at[0, pl.ds(_STRIDE * k, 1), :],
            zsems.at[k])
        cp.start()
        zcopies.append(cp)
    for cp in copies:
        cp.wait()
    for cp in zcopies:
        cp.wait()


def kernel(data):
    return pl.pallas_call(
        _dma_kernel,
        in_specs=[pl.BlockSpec(memory_space=pltpu.MemorySpace.HBM)],
        out_specs=pl.BlockSpec(memory_space=pltpu.MemorySpace.HBM),
        out_shape=jax.ShapeDtypeStruct(data.shape, data.dtype),
        scratch_shapes=[
            pltpu.MemorySpace.VMEM((8, 128), jnp.float32),
            pltpu.SemaphoreType.DMA((_CHUNKS1 + _NIDS,)),
            pltpu.SemaphoreType.DMA((_NIDS,)),
        ],
    )(data)


# speedup vs baseline: 49.1566x; 49.1566x over previous
import jax
import jax.numpy as jnp
from jax.experimental import pallas as pl

_B = 25000
_STRIDE = 3000
_NIDS = 64


def _copy_kernel(x_ref, o_ref):
    i = pl.program_id(0)
    j = pl.program_id(1)
    o_ref[0] = x_ref[0]
    for k in range(_NIDS):
        rid = k * _STRIDE

        @pl.when((i == 0) & (j == rid // _B))
        def _zero_row(rid=rid):
            o_ref[0, rid % _B, :] = jnp.zeros((128,), jnp.float32)


def kernel(data):
    n = data.shape[1]
    return pl.pallas_call(
        _copy_kernel,
        grid=(data.shape[0], n // _B),
        in_specs=[pl.BlockSpec((1, _B, 128), lambda i, j: (i, j, 0))],
        out_specs=pl.BlockSpec((1, _B, 128), lambda i, j: (i, j, 0)),
        out_shape=jax.ShapeDtypeStruct(data.shape, data.dtype),
    )(data)
